# Initial kernel scaffold; baseline (speedup 1.0000x reference)
import jax
import jax.numpy as jnp
from jax.experimental import pallas as pl


def kernel(input_ids, text_table, audio_table, audio_tokens_offsets):
    raise NotImplementedError("write your pallas kernel here")


# speedup vs baseline: 1.2869x; 1.2869x over previous
import functools

import jax
import jax.numpy as jnp
from jax import lax
from jax.experimental import pallas as pl
from jax.experimental.pallas import tpu as pltpu
from jax.experimental.pallas import tpu_sc as plsc

HIDDEN = 2048
NUM_CB = 32
L = 16
NWORK = 32
TOK = 4096
TPW = TOK // NWORK
GRP = 8
NGRP = TPW // GRP
NHID = HIDDEN // L
UR = 8
UPT = NUM_CB // UR
NUNIT = TPW * UPT


def _sc_embed(ids_audio, ids_text, text_table, audio_table, offsets):
    mesh = plsc.VectorSubcoreMesh(core_axis_name="c", subcore_axis_name="s")

    @functools.partial(
        pl.kernel,
        mesh=mesh,
        out_type=jax.ShapeDtypeStruct((TOK, HIDDEN), jnp.float32),
        scratch_types=[
            pltpu.VMEM((TPW * NUM_CB,), jnp.int32),
            pltpu.VMEM((TPW,), jnp.int32),
            pltpu.VMEM((NUM_CB,), jnp.int32),
            pltpu.VMEM((2, GRP, HIDDEN), jnp.float32),
            pltpu.VMEM((UR, HIDDEN), jnp.float32),
            pltpu.VMEM((UR, HIDDEN), jnp.float32),
            pltpu.VMEM((2, 1, HIDDEN), jnp.float32),
            pltpu.SemaphoreType.DMA,
            pltpu.SemaphoreType.DMA,
            pltpu.SemaphoreType.DMA,
            pltpu.SemaphoreType.DMA,
            pltpu.SemaphoreType.DMA,
            pltpu.SemaphoreType.DMA,
        ],
    )
    def body(ids_audio_h, ids_text_h, ttab_h, atab_h, offs_h, out_h,
             aidx_v, tid_v, offs_v, tb, b0, b1, acc,
             sem_a0, sem_a1, sem_t0, sem_t1, sem_o0, sem_o1):
        wid = lax.axis_index("s") * 2 + lax.axis_index("c")
        base = wid * TPW

        pltpu.sync_copy(ids_audio_h.at[pl.ds(base * NUM_CB, TPW * NUM_CB)], aidx_v)
        pltpu.sync_copy(ids_text_h.at[pl.ds(base, TPW)], tid_v)
        pltpu.sync_copy(offs_h, offs_v)

        pltpu.async_copy(ttab_h.at[tid_v.at[pl.ds(0, GRP)]], tb.at[0], sem_t0)

        zeros = jnp.zeros((L,), jnp.int32)
        offs01 = offs_v[pl.ds(0, L)]
        offs23 = offs_v[pl.ds(L, L)]

        def cidx(t, carry):
            tok01 = aidx_v[pl.ds(NUM_CB * t, L)]
            tok23 = aidx_v[pl.ds(NUM_CB * t + L, L)]
            aidx_v[pl.ds(NUM_CB * t, L)] = jnp.where(tok01 == 0, zeros, tok01 + offs01)
            aidx_v[pl.ds(NUM_CB * t + L, L)] = jnp.where(tok23 == 0, zeros, tok23 + offs23)
            return carry
        lax.fori_loop(0, TPW, cidx, 0)

        pltpu.async_copy(atab_h.at[aidx_v.at[pl.ds(0, UR)]], b0, sem_a0)

        bufs = (b0, b1)
        sems = (sem_a0, sem_a1)

        def tok_body(t, carry):
            g = t // GRP
            gl = t % GRP
            po = t % 2
            is_gs = gl == 0
            even_g = (g % 2) == 0

            @pl.when(jnp.logical_and(is_gs, even_g))
            def _():
                pltpu.make_async_copy(ttab_h.at[pl.ds(0, GRP)], tb.at[0], sem_t0).wait()

            @pl.when(jnp.logical_and(is_gs, jnp.logical_not(even_g)))
            def _():
                pltpu.make_async_copy(ttab_h.at[pl.ds(0, GRP)], tb.at[1], sem_t1).wait()

            @pl.when(jnp.logical_and(is_gs, jnp.logical_and(even_g, g + 1 < NGRP)))
            def _():
                pltpu.async_copy(
                    ttab_h.at[tid_v.at[pl.ds((g + 1) * GRP, GRP)]], tb.at[1], sem_t1)

            @pl.when(jnp.logical_and(is_gs,
                                     jnp.logical_and(jnp.logical_not(even_g),
                                                     g + 1 < NGRP)))
            def _():
                pltpu.async_copy(
                    ttab_h.at[tid_v.at[pl.ds((g + 1) * GRP, GRP)]], tb.at[0], sem_t0)

            @pl.when(jnp.logical_and(po == 0, t >= 2))
            def _():
                pltpu.make_async_copy(out_h.at[pl.ds(0, 1)], acc.at[0], sem_o0).wait()

            @pl.when(jnp.logical_and(po == 1, t >= 2))
            def _():
                pltpu.make_async_copy(out_h.at[pl.ds(0, 1)], acc.at[1], sem_o1).wait()

            u = UPT * t
            for h in range(UPT):
                cur, nxt = bufs[h % 2], bufs[(h + 1) % 2]
                cur_s, nxt_s = sems[h % 2], sems[(h + 1) % 2]
                if h < UPT - 1:
                    pltpu.async_copy(
                        atab_h.at[aidx_v.at[pl.ds((u + h + 1) * UR, UR)]], nxt, nxt_s)
                else:
                    @pl.when(t + 1 < TPW)
                    def _():
                        pltpu.async_copy(
                            atab_h.at[aidx_v.at[pl.ds((u + UPT) * UR, UR)]], nxt, nxt_s)
                pltpu.make_async_copy(atab_h.at[pl.ds(0, UR)], cur, cur_s).wait()

                if h == 0:
                    def acc_first(c, carry2):
                        cs = pl.ds(c * L, L)
                        v = tb[g % 2, gl, cs]
                        for r in range(UR):
                            v = v + cur[r, cs]
                        acc[po, 0, cs] = v
                        return carry2
                    lax.fori_loop(0, NHID, acc_first, 0)
                else:
                    def acc_rest(c, carry2, cur=cur):
                        cs = pl.ds(c * L, L)
                        v = acc[po, 0, cs]
                        for r in range(UR):
                            v = v + cur[r, cs]
                        acc[po, 0, cs] = v
                        return carry2
                    lax.fori_loop(0, NHID, acc_rest, 0)

            @pl.when(po == 0)
            def _():
                pltpu.async_copy(acc.at[0], out_h.at[pl.ds(base + t, 1)], sem_o0)

            @pl.when(po == 1)
            def _():
                pltpu.async_copy(acc.at[1], out_h.at[pl.ds(base + t, 1)], sem_o1)

            return carry
        lax.fori_loop(0, TPW, tok_body, 0)

        pltpu.make_async_copy(out_h.at[pl.ds(0, 1)], acc.at[0], sem_o0).wait()
        pltpu.make_async_copy(out_h.at[pl.ds(0, 1)], acc.at[1], sem_o1).wait()

    return body(ids_audio, ids_text, text_table, audio_table, offsets)


def kernel(input_ids, text_table, audio_table, audio_tokens_offsets):
    b, s, _ = input_ids.shape
    ids = input_ids.reshape(b * s, NUM_CB + 1).astype(jnp.int32)
    ids_audio = ids[:, :NUM_CB].reshape(TOK * NUM_CB)
    ids_text = ids[:, NUM_CB]
    offs = audio_tokens_offsets.astype(jnp.int32)
    out = _sc_embed(ids_audio, ids_text, text_table, audio_table, offs)
    return out.reshape(b, s, HIDDEN)


# speedup vs baseline: 1.5561x; 1.2092x over previous
import functools

import jax
import jax.numpy as jnp
from jax import lax
from jax.experimental import pallas as pl
from jax.experimental.pallas import tpu as pltpu
from jax.experimental.pallas import tpu_sc as plsc

HIDDEN = 2048
NUM_CB = 32
L = 16
NWORK = 32
TOK = 4096
TPW = TOK // NWORK
GRP = 8
NGRP = TPW // GRP
NHID = HIDDEN // L
UR = 16
UPT = NUM_CB // UR


def _sc_embed(ids_audio, ids_text, text_table, audio_table, offsets):
    mesh = plsc.VectorSubcoreMesh(core_axis_name="c", subcore_axis_name="s")

    @functools.partial(
        pl.kernel,
        mesh=mesh,
        out_type=jax.ShapeDtypeStruct((TOK, HIDDEN), jnp.float32),
        scratch_types=[
            pltpu.VMEM((TPW * NUM_CB,), jnp.int32),
            pltpu.VMEM((TPW,), jnp.int32),
            pltpu.VMEM((NUM_CB,), jnp.int32),
            pltpu.VMEM((GRP, HIDDEN), jnp.float32),
            pltpu.VMEM((UR, HIDDEN), jnp.float32),
            pltpu.VMEM((UR, HIDDEN), jnp.float32),
            pltpu.VMEM((2, 1, HIDDEN), jnp.float32),
            pltpu.SemaphoreType.DMA,
            pltpu.SemaphoreType.DMA,
            pltpu.SemaphoreType.DMA,
            pltpu.SemaphoreType.DMA,
            pltpu.SemaphoreType.DMA,
        ],
    )
    def body(ids_audio_h, ids_text_h, ttab_h, atab_h, offs_h, out_h,
             aidx_v, tid_v, offs_v, tb, b0, b1, acc,
             sem_a0, sem_a1, sem_t, sem_o0, sem_o1):
        wid = lax.axis_index("s") * 2 + lax.axis_index("c")
        base = wid * TPW

        pltpu.sync_copy(ids_audio_h.at[pl.ds(base * NUM_CB, TPW * NUM_CB)], aidx_v)
        pltpu.sync_copy(ids_text_h.at[pl.ds(base, TPW)], tid_v)
        pltpu.sync_copy(offs_h, offs_v)

        pltpu.async_copy(ttab_h.at[tid_v.at[pl.ds(0, GRP)]], tb, sem_t)

        zeros = jnp.zeros((L,), jnp.int32)
        offs01 = offs_v[pl.ds(0, L)]
        offs23 = offs_v[pl.ds(L, L)]

        def cidx(t, carry):
            tok01 = aidx_v[pl.ds(NUM_CB * t, L)]
            tok23 = aidx_v[pl.ds(NUM_CB * t + L, L)]
            aidx_v[pl.ds(NUM_CB * t, L)] = jnp.where(tok01 == 0, zeros, tok01 + offs01)
            aidx_v[pl.ds(NUM_CB * t + L, L)] = jnp.where(tok23 == 0, zeros, tok23 + offs23)
            return carry
        lax.fori_loop(0, TPW, cidx, 0)

        pltpu.async_copy(atab_h.at[aidx_v.at[pl.ds(0, UR)]], b0, sem_a0)

        def tok_body(t, carry):
            g = t // GRP
            gl = t % GRP
            po = t % 2

            @pl.when(gl == 0)
            def _():
                pltpu.make_async_copy(ttab_h.at[pl.ds(0, GRP)], tb, sem_t).wait()

            @pl.when(jnp.logical_and(po == 0, t >= 2))
            def _():
                pltpu.make_async_copy(out_h.at[pl.ds(0, 1)], acc.at[0], sem_o0).wait()

            @pl.when(jnp.logical_and(po == 1, t >= 2))
            def _():
                pltpu.make_async_copy(out_h.at[pl.ds(0, 1)], acc.at[1], sem_o1).wait()

            u = UPT * t
            pltpu.async_copy(atab_h.at[aidx_v.at[pl.ds((u + 1) * UR, UR)]], b1, sem_a1)
            pltpu.make_async_copy(atab_h.at[pl.ds(0, UR)], b0, sem_a0).wait()

            def acc0(c, carry2):
                for k in range(2):
                    cs = pl.ds((2 * c + k) * L, L)
                    v = tb[gl, cs]
                    for r in range(UR):
                        v = v + b0[r, cs]
                    acc[po, 0, cs] = v
                return carry2
            lax.fori_loop(0, NHID // 2, acc0, 0)

            @pl.when(jnp.logical_and(gl == GRP - 1, g + 1 < NGRP))
            def _():
                pltpu.async_copy(
                    ttab_h.at[tid_v.at[pl.ds((g + 1) * GRP, GRP)]], tb, sem_t)

            @pl.when(t + 1 < TPW)
            def _():
                pltpu.async_copy(
                    atab_h.at[aidx_v.at[pl.ds((u + 2) * UR, UR)]], b0, sem_a0)

            pltpu.make_async_copy(atab_h.at[pl.ds(0, UR)], b1, sem_a1).wait()

            def acc1(c, carry2):
                for k in range(2):
                    cs = pl.ds((2 * c + k) * L, L)
                    v = acc[po, 0, cs]
                    for r in range(UR):
                        v = v + b1[r, cs]
                    acc[po, 0, cs] = v
                return carry2
            lax.fori_loop(0, NHID // 2, acc1, 0)

            @pl.when(po == 0)
            def _():
                pltpu.async_copy(acc.at[0], out_h.at[pl.ds(base + t, 1)], sem_o0)

            @pl.when(po == 1)
            def _():
                pltpu.async_copy(acc.at[1], out_h.at[pl.ds(base + t, 1)], sem_o1)

            return carry
        lax.fori_loop(0, TPW, tok_body, 0)

        pltpu.make_async_copy(out_h.at[pl.ds(0, 1)], acc.at[0], sem_o0).wait()
        pltpu.make_async_copy(out_h.at[pl.ds(0, 1)], acc.at[1], sem_o1).wait()

    return body(ids_audio, ids_text, text_table, audio_table, offsets)


def kernel(input_ids, text_table, audio_table, audio_tokens_offsets):
    b, s, _ = input_ids.shape
    ids = input_ids.reshape(b * s, NUM_CB + 1).astype(jnp.int32)
    ids_audio = ids[:, :NUM_CB].reshape(TOK * NUM_CB)
    ids_text = ids[:, NUM_CB]
    offs = audio_tokens_offsets.astype(jnp.int32)
    out = _sc_embed(ids_audio, ids_text, text_table, audio_table, offs)
    return out.reshape(b, s, HIDDEN)


# speedup vs baseline: 1.9253x; 1.2373x over previous
import functools

import jax
import jax.numpy as jnp
from jax import lax
from jax.experimental import pallas as pl
from jax.experimental.pallas import tpu as pltpu
from jax.experimental.pallas import tpu_sc as plsc

HIDDEN = 2048
NUM_CB = 32
L = 16
NWORK = 32
TOK = 4096
TPW = TOK // NWORK
GRP = 8
NGRP = TPW // GRP
NHID = HIDDEN // L
UR = 16
UPT = NUM_CB // UR


def _sc_embed(ids_audio, ids_text, text_table, audio_table, offsets):
    mesh = plsc.VectorSubcoreMesh(core_axis_name="c", subcore_axis_name="s")

    @functools.partial(
        pl.kernel,
        mesh=mesh,
        out_type=jax.ShapeDtypeStruct((TOK, HIDDEN), jnp.float32),
        scratch_types=[
            pltpu.VMEM((TPW * NUM_CB,), jnp.int32),
            pltpu.VMEM((TPW,), jnp.int32),
            pltpu.VMEM((NUM_CB,), jnp.int32),
            pltpu.VMEM((GRP, HIDDEN), jnp.float32),
            pltpu.VMEM((UR, HIDDEN), jnp.float32),
            pltpu.VMEM((UR, HIDDEN), jnp.float32),
            pltpu.VMEM((2, 1, HIDDEN), jnp.float32),
            pltpu.SemaphoreType.DMA,
            pltpu.SemaphoreType.DMA,
            pltpu.SemaphoreType.DMA,
            pltpu.SemaphoreType.DMA,
            pltpu.SemaphoreType.DMA,
        ],
    )
    def body(ids_audio_h, ids_text_h, ttab_h, atab_h, offs_h, out_h,
             aidx_v, tid_v, offs_v, tb, b0, b1, acc,
             sem_a0, sem_a1, sem_t, sem_o0, sem_o1):
        wid = lax.axis_index("s") * 2 + lax.axis_index("c")
        base = wid * TPW

        pltpu.sync_copy(ids_audio_h.at[pl.ds(base * NUM_CB, TPW * NUM_CB)], aidx_v)
        pltpu.sync_copy(ids_text_h.at[pl.ds(base, TPW)], tid_v)
        pltpu.sync_copy(offs_h, offs_v)

        pltpu.async_copy(ttab_h.at[tid_v.at[pl.ds(0, GRP)]], tb, sem_t)

        zeros = jnp.zeros((L,), jnp.int32)
        offs01 = offs_v[pl.ds(0, L)]
        offs23 = offs_v[pl.ds(L, L)]

        def cidx(t, carry):
            tok01 = aidx_v[pl.ds(NUM_CB * t, L)]
            tok23 = aidx_v[pl.ds(NUM_CB * t + L, L)]
            aidx_v[pl.ds(NUM_CB * t, L)] = jnp.where(tok01 == 0, zeros, tok01 + offs01)
            aidx_v[pl.ds(NUM_CB * t + L, L)] = jnp.where(tok23 == 0, zeros, tok23 + offs23)
            return carry
        lax.fori_loop(0, TPW, cidx, 0)

        pltpu.async_copy(atab_h.at[aidx_v.at[pl.ds(0, UR)]], b0, sem_a0)

        def tok_body(t, carry):
            g = t // GRP
            gl = t % GRP
            po = t % 2

            @pl.when(gl == 0)
            def _():
                pltpu.make_async_copy(ttab_h.at[pl.ds(0, GRP)], tb, sem_t).wait()

            @pl.when(jnp.logical_and(po == 0, t >= 2))
            def _():
                pltpu.make_async_copy(out_h.at[pl.ds(0, 1)], acc.at[0], sem_o0).wait()

            @pl.when(jnp.logical_and(po == 1, t >= 2))
            def _():
                pltpu.make_async_copy(out_h.at[pl.ds(0, 1)], acc.at[1], sem_o1).wait()

            u = UPT * t
            pltpu.async_copy(atab_h.at[aidx_v.at[pl.ds((u + 1) * UR, UR)]], b1, sem_a1)
            pltpu.make_async_copy(atab_h.at[pl.ds(0, UR)], b0, sem_a0).wait()

            def acc0(c, carry2):
                for k in range(2):
                    cs = pl.ds((2 * c + k) * L, L)
                    vals = [tb[gl, cs]] + [b0[r, cs] for r in range(UR)]
                    while len(vals) > 1:
                        vals = [a + b for a, b in zip(vals[::2], vals[1::2])] \
                            + ([vals[-1]] if len(vals) % 2 else [])
                    acc[po, 0, cs] = vals[0]
                return carry2
            lax.fori_loop(0, NHID // 2, acc0, 0)

            @pl.when(jnp.logical_and(gl == GRP - 1, g + 1 < NGRP))
            def _():
                pltpu.async_copy(
                    ttab_h.at[tid_v.at[pl.ds((g + 1) * GRP, GRP)]], tb, sem_t)

            @pl.when(t + 1 < TPW)
            def _():
                pltpu.async_copy(
                    atab_h.at[aidx_v.at[pl.ds((u + 2) * UR, UR)]], b0, sem_a0)

            pltpu.make_async_copy(atab_h.at[pl.ds(0, UR)], b1, sem_a1).wait()

            def acc1(c, carry2):
                for k in range(2):
                    cs = pl.ds((2 * c + k) * L, L)
                    vals = [acc[po, 0, cs]] + [b1[r, cs] for r in range(UR)]
                    while len(vals) > 1:
                        vals = [a + b for a, b in zip(vals[::2], vals[1::2])] \
                            + ([vals[-1]] if len(vals) % 2 else [])
                    acc[po, 0, cs] = vals[0]
                return carry2
            lax.fori_loop(0, NHID // 2, acc1, 0)

            @pl.when(po == 0)
            def _():
                pltpu.async_copy(acc.at[0], out_h.at[pl.ds(base + t, 1)], sem_o0)

            @pl.when(po == 1)
            def _():
                pltpu.async_copy(acc.at[1], out_h.at[pl.ds(base + t, 1)], sem_o1)

            return carry
        lax.fori_loop(0, TPW, tok_body, 0)

        pltpu.make_async_copy(out_h.at[pl.ds(0, 1)], acc.at[0], sem_o0).wait()
        pltpu.make_async_copy(out_h.at[pl.ds(0, 1)], acc.at[1], sem_o1).wait()

    return body(ids_audio, ids_text, text_table, audio_table, offsets)


def kernel(input_ids, text_table, audio_table, audio_tokens_offsets):
    b, s, _ = input_ids.shape
    ids = input_ids.reshape(b * s, NUM_CB + 1).astype(jnp.int32)
    ids_audio = ids[:, :NUM_CB].reshape(TOK * NUM_CB)
    ids_text = ids[:, NUM_CB]
    offs = audio_tokens_offsets.astype(jnp.int32)
    out = _sc_embed(ids_audio, ids_text, text_table, audio_table, offs)
    return out.reshape(b, s, HIDDEN)
